# baseline (speedup 1.0000x reference)
import functools

import jax
import jax.numpy as jnp
from jax import lax
from jax.experimental import pallas as pl
from jax.experimental.pallas import tpu as pltpu
from jax.experimental.pallas import tpu_sc as plsc

N = 10000
E = 320000
D = 128

NC = 2
NS = 16
NW = NC * NS
EPW = E // NW
C = 40
CH = EPW // C
VPR = D // 16
RPT = 624
TAIL = N - NS * RPT

IA = 6
GA = 4
EA = 1
RI = 8
RG = 5
RE = 2
RM = 2


def _sc_body(x_hbm, src_hbm, dst_hbm, ea_hbm, out_hbm,
             sidx, didx, xr, ea, msg, acc, isem, gsem, esem, ssem):
    cid = lax.axis_index("c")
    sid = lax.axis_index("s")
    wid = cid * NS + sid
    base_e = wid * EPW

    z0 = xr.at[0]

    def _zero_row(r, carry):
        for j in range(VPR):
            xr[0, r, pl.ds(j * 16, 16)] = jnp.zeros((16,), jnp.float32)
        return carry
    lax.fori_loop(0, C, _zero_row, 0)
    base_r = sid * RPT
    for t in range(RPT // C):
        pltpu.sync_copy(z0, acc.at[pl.ds(base_r + t * C, C)])
    rem = RPT % C
    if rem:
        pltpu.sync_copy(z0.at[pl.ds(0, rem)],
                        acc.at[pl.ds(base_r + (RPT // C) * C, rem)])

    @pl.when(sid == NS - 1)
    def _zero_tail():
        pltpu.sync_copy(z0.at[pl.ds(0, TAIL)],
                        acc.at[pl.ds(NS * RPT, TAIL)])

    plsc.subcore_barrier()

    def fire_idx(k):
        s = lax.rem(k, RI)
        pltpu.async_copy(src_hbm.at[pl.ds(base_e + k * C, C)], sidx.at[s],
                         isem.at[s])
        pltpu.async_copy(dst_hbm.at[pl.ds(base_e + k * C, C)], didx.at[s],
                         isem.at[s])

    def wait_idx(k):
        s = lax.rem(k, RI)
        pltpu.make_async_copy(src_hbm.at[pl.ds(base_e, C)], sidx.at[s],
                              isem.at[s]).wait()
        pltpu.make_async_copy(dst_hbm.at[pl.ds(base_e, C)], didx.at[s],
                              isem.at[s]).wait()

    def fire_gather(k):
        s = lax.rem(k, RG)
        pltpu.async_copy(x_hbm.at[sidx.at[lax.rem(k, RI)]], xr.at[s],
                         gsem.at[s])

    def wait_gather(k):
        s = lax.rem(k, RG)
        pltpu.make_async_copy(x_hbm.at[sidx.at[0]], xr.at[s],
                              gsem.at[s]).wait()

    def fire_ea(k):
        s = lax.rem(k, RE)
        pltpu.async_copy(ea_hbm.at[pl.ds(base_e + k * C, C)], ea.at[s],
                         esem.at[s])

    def wait_ea(k):
        s = lax.rem(k, RE)
        pltpu.make_async_copy(ea_hbm.at[pl.ds(base_e, C)], ea.at[s],
                              esem.at[s]).wait()

    def compute(k):
        sg = lax.rem(k, RG)
        se = lax.rem(k, RE)
        sm = lax.rem(k, RM)

        @plsc.parallel_loop(0, C, step=1, unroll=2)
        def _row(r):
            for j in range(VPR):
                sl = pl.ds(j * 16, 16)
                msg[sm, r, sl] = jnp.maximum(
                    xr[sg, r, sl] + ea[se, r, sl], 0.0)

    def fire_scat(k):
        sm = lax.rem(k, RM)
        pltpu.async_copy(msg.at[sm], acc.at[didx.at[lax.rem(k, RI)]],
                         ssem.at[sm], add=True)

    def wait_scat(k):
        sm = lax.rem(k, RM)
        pltpu.make_async_copy(msg.at[sm], acc.at[didx.at[0]],
                              ssem.at[sm]).wait()

    for c0 in range(IA):
        fire_idx(c0)
    for c0 in range(GA):
        wait_idx(c0)
        fire_gather(c0)
    for c0 in range(EA):
        fire_ea(c0)

    def _step(c, carry):
        @pl.when(c + IA < CH)
        def _():
            fire_idx(c + IA)

        @pl.when(c + GA < CH)
        def _():
            wait_idx(c + GA)
            fire_gather(c + GA)

        @pl.when(c + EA < CH)
        def _():
            fire_ea(c + EA)
        wait_gather(c)
        wait_ea(c)

        @pl.when(c >= RM)
        def _():
            wait_scat(c - RM)
        compute(c)
        fire_scat(c)
        return carry

    lax.fori_loop(0, CH, _step, 0)
    wait_scat(CH - 2)
    wait_scat(CH - 1)

    plsc.subcore_barrier()
    pltpu.sync_copy(acc.at[pl.ds(base_r, RPT)],
                    out_hbm.at[cid, pl.ds(base_r, RPT)])

    @pl.when(sid == NS - 1)
    def _copy_tail():
        pltpu.sync_copy(acc.at[pl.ds(NS * RPT, TAIL)],
                        out_hbm.at[cid, pl.ds(NS * RPT, TAIL)])


_sc_aggr = functools.partial(
    pl.kernel,
    out_type=jax.ShapeDtypeStruct((NC, N, D), jnp.float32),
    mesh=plsc.VectorSubcoreMesh(core_axis_name="c", subcore_axis_name="s"),
    scratch_types=[
        pltpu.VMEM((RI, C), jnp.int32),
        pltpu.VMEM((RI, C), jnp.int32),
        pltpu.VMEM((RG, C, D), jnp.float32),
        pltpu.VMEM((RE, C, D), jnp.float32),
        pltpu.VMEM((RM, C, D), jnp.float32),
        pltpu.VMEM_SHARED((N, D), jnp.float32),
        pltpu.SemaphoreType.DMA((RI,)),
        pltpu.SemaphoreType.DMA((RG,)),
        pltpu.SemaphoreType.DMA((RE,)),
        pltpu.SemaphoreType.DMA((RM,)),
    ],
)(_sc_body)


def _tc_body(x_ref, a0_ref, a1_ref, w1_ref, b1_ref, w2_ref, b2_ref,
             g_ref, be_ref, o_ref):
    xb = x_ref[...]
    z = xb + a0_ref[...] + a1_ref[...]
    h1 = jnp.dot(z, w1_ref[...], preferred_element_type=jnp.float32)
    h1 = jnp.maximum(h1 + b1_ref[...], 0.0)
    h = jnp.dot(h1, w2_ref[...], preferred_element_type=jnp.float32)
    h = h + b2_ref[...]
    r = xb + jnp.maximum(h, 0.0)
    m = jnp.mean(r, axis=1, keepdims=True)
    cdev = r - m
    v = jnp.mean(cdev * cdev, axis=1, keepdims=True)
    o_ref[...] = cdev * lax.rsqrt(v + 1e-5) * g_ref[...] + be_ref[...]


BN = 1000


def _tc_mlp_ln(x, a0, a1, w1, b1, w2, b2, gamma, beta):
    row_spec = pl.BlockSpec((BN, D), lambda i: (i, 0))
    full_spec = pl.BlockSpec((D, D), lambda i: (0, 0))
    vec_spec = pl.BlockSpec((1, D), lambda i: (0, 0))
    return pl.pallas_call(
        _tc_body,
        grid=(N // BN,),
        in_specs=[row_spec, row_spec, row_spec, full_spec, vec_spec,
                  full_spec, vec_spec, vec_spec, vec_spec],
        out_specs=row_spec,
        out_shape=jax.ShapeDtypeStruct((N, D), jnp.float32),
    )(x, a0, a1, w1, b1.reshape(1, D), w2, b2.reshape(1, D),
      gamma.reshape(1, D), beta.reshape(1, D))


def kernel(x, edge_index, edge_attr, W1, b1, W2, b2, gamma, beta):
    src = edge_index[0].astype(jnp.int32)
    dst = edge_index[1].astype(jnp.int32)
    parts = _sc_aggr(x, src, dst, edge_attr)
    return _tc_mlp_ln(x, parts[0], parts[1], W1, b1, W2, b2, gamma, beta)


# speedup vs baseline: 1.1529x; 1.1529x over previous
import functools

import jax
import jax.numpy as jnp
from jax import lax
from jax.experimental import pallas as pl
from jax.experimental.pallas import tpu as pltpu
from jax.experimental.pallas import tpu_sc as plsc

N = 10000
E = 320000
D = 128

NC = 2
NS = 16
NW = NC * NS
EPW = E // NW
C = 40
CH = EPW // C
VPR = D // 16
RPT = 624
TAIL = N - NS * RPT

IA = 5
GA = 3
EA = 2
RI = 8
RG = 4
RE = 3
RM = 2


def _sc_body(x_hbm, src_hbm, dst_hbm, ea_hbm, out_hbm,
             sidx, didx, xr, ea, msg, acc, isem, gsem, esem, ssem):
    cid = lax.axis_index("c")
    sid = lax.axis_index("s")
    wid = cid * NS + sid
    base_e = wid * EPW

    z0 = xr.at[0]

    def _zero_row(r, carry):
        for j in range(VPR):
            xr[0, r, pl.ds(j * 16, 16)] = jnp.zeros((16,), jnp.float32)
        return carry
    lax.fori_loop(0, C, _zero_row, 0)
    base_r = sid * RPT
    for t in range(RPT // C):
        pltpu.sync_copy(z0, acc.at[pl.ds(base_r + t * C, C)])
    rem = RPT % C
    if rem:
        pltpu.sync_copy(z0.at[pl.ds(0, rem)],
                        acc.at[pl.ds(base_r + (RPT // C) * C, rem)])

    @pl.when(sid == NS - 1)
    def _zero_tail():
        pltpu.sync_copy(z0.at[pl.ds(0, TAIL)],
                        acc.at[pl.ds(NS * RPT, TAIL)])

    plsc.subcore_barrier()

    def fire_idx(k):
        s = lax.rem(k, RI)
        pltpu.async_copy(src_hbm.at[pl.ds(base_e + k * C, C)], sidx.at[s],
                         isem.at[s])
        pltpu.async_copy(dst_hbm.at[pl.ds(base_e + k * C, C)], didx.at[s],
                         isem.at[s])

    def wait_idx(k):
        s = lax.rem(k, RI)
        pltpu.make_async_copy(src_hbm.at[pl.ds(base_e, C)], sidx.at[s],
                              isem.at[s]).wait()
        pltpu.make_async_copy(dst_hbm.at[pl.ds(base_e, C)], didx.at[s],
                              isem.at[s]).wait()

    def fire_gather(k):
        s = lax.rem(k, RG)
        pltpu.async_copy(x_hbm.at[sidx.at[lax.rem(k, RI)]], xr.at[s],
                         gsem.at[s])

    def wait_gather(k):
        s = lax.rem(k, RG)
        pltpu.make_async_copy(x_hbm.at[sidx.at[0]], xr.at[s],
                              gsem.at[s]).wait()

    def fire_ea(k):
        s = lax.rem(k, RE)
        pltpu.async_copy(ea_hbm.at[pl.ds(base_e + k * C, C)], ea.at[s],
                         esem.at[s])

    def wait_ea(k):
        s = lax.rem(k, RE)
        pltpu.make_async_copy(ea_hbm.at[pl.ds(base_e, C)], ea.at[s],
                              esem.at[s]).wait()

    def compute(k):
        sg = lax.rem(k, RG)
        se = lax.rem(k, RE)
        sm = lax.rem(k, RM)

        @plsc.parallel_loop(0, C, step=1, unroll=2)
        def _row(r):
            for j in range(VPR):
                sl = pl.ds(j * 16, 16)
                msg[sm, r, sl] = jnp.maximum(
                    xr[sg, r, sl] + ea[se, r, sl], 0.0)

    def fire_scat(k):
        sm = lax.rem(k, RM)
        pltpu.async_copy(msg.at[sm], acc.at[didx.at[lax.rem(k, RI)]],
                         ssem.at[sm], add=True)

    def wait_scat(k):
        sm = lax.rem(k, RM)
        pltpu.make_async_copy(msg.at[sm], acc.at[didx.at[0]],
                              ssem.at[sm]).wait()

    for c0 in range(IA):
        fire_idx(c0)
    for c0 in range(GA):
        wait_idx(c0)
        fire_gather(c0)
    for c0 in range(EA):
        fire_ea(c0)

    def _step(c, carry):
        @pl.when(c + IA < CH)
        def _():
            fire_idx(c + IA)

        @pl.when(c + GA < CH)
        def _():
            wait_idx(c + GA)
            fire_gather(c + GA)

        @pl.when(c + EA < CH)
        def _():
            fire_ea(c + EA)
        wait_gather(c)
        wait_ea(c)

        @pl.when(c >= RM)
        def _():
            wait_scat(c - RM)
        compute(c)
        fire_scat(c)
        return carry

    lax.fori_loop(0, CH, _step, 0)
    wait_scat(CH - 2)
    wait_scat(CH - 1)

    plsc.subcore_barrier()
    pltpu.sync_copy(acc.at[pl.ds(base_r, RPT)],
                    out_hbm.at[cid, pl.ds(base_r, RPT)])

    @pl.when(sid == NS - 1)
    def _copy_tail():
        pltpu.sync_copy(acc.at[pl.ds(NS * RPT, TAIL)],
                        out_hbm.at[cid, pl.ds(NS * RPT, TAIL)])


_sc_aggr = functools.partial(
    pl.kernel,
    out_type=jax.ShapeDtypeStruct((NC, N, D), jnp.float32),
    mesh=plsc.VectorSubcoreMesh(core_axis_name="c", subcore_axis_name="s"),
    scratch_types=[
        pltpu.VMEM((RI, C), jnp.int32),
        pltpu.VMEM((RI, C), jnp.int32),
        pltpu.VMEM((RG, C, D), jnp.float32),
        pltpu.VMEM((RE, C, D), jnp.float32),
        pltpu.VMEM((RM, C, D), jnp.float32),
        pltpu.VMEM_SHARED((N, D), jnp.float32),
        pltpu.SemaphoreType.DMA((RI,)),
        pltpu.SemaphoreType.DMA((RG,)),
        pltpu.SemaphoreType.DMA((RE,)),
        pltpu.SemaphoreType.DMA((RM,)),
    ],
)(_sc_body)


def _tc_body(x_ref, a0_ref, a1_ref, w1_ref, b1_ref, w2_ref, b2_ref,
             g_ref, be_ref, o_ref):
    xb = x_ref[...]
    z = xb + a0_ref[...] + a1_ref[...]
    h1 = jnp.dot(z, w1_ref[...], preferred_element_type=jnp.float32)
    h1 = jnp.maximum(h1 + b1_ref[...], 0.0)
    h = jnp.dot(h1, w2_ref[...], preferred_element_type=jnp.float32)
    h = h + b2_ref[...]
    r = xb + jnp.maximum(h, 0.0)
    m = jnp.mean(r, axis=1, keepdims=True)
    cdev = r - m
    v = jnp.mean(cdev * cdev, axis=1, keepdims=True)
    o_ref[...] = cdev * lax.rsqrt(v + 1e-5) * g_ref[...] + be_ref[...]


BN = 1000


def _tc_mlp_ln(x, a0, a1, w1, b1, w2, b2, gamma, beta):
    row_spec = pl.BlockSpec((BN, D), lambda i: (i, 0))
    full_spec = pl.BlockSpec((D, D), lambda i: (0, 0))
    vec_spec = pl.BlockSpec((1, D), lambda i: (0, 0))
    return pl.pallas_call(
        _tc_body,
        grid=(N // BN,),
        in_specs=[row_spec, row_spec, row_spec, full_spec, vec_spec,
                  full_spec, vec_spec, vec_spec, vec_spec],
        out_specs=row_spec,
        out_shape=jax.ShapeDtypeStruct((N, D), jnp.float32),
    )(x, a0, a1, w1, b1.reshape(1, D), w2, b2.reshape(1, D),
      gamma.reshape(1, D), beta.reshape(1, D))


def kernel(x, edge_index, edge_attr, W1, b1, W2, b2, gamma, beta):
    src = edge_index[0].astype(jnp.int32)
    dst = edge_index[1].astype(jnp.int32)
    parts = _sc_aggr(x, src, dst, edge_attr)
    return _tc_mlp_ln(x, parts[0], parts[1], W1, b1, W2, b2, gamma, beta)


# speedup vs baseline: 1.1626x; 1.0084x over previous
import functools

import jax
import jax.numpy as jnp
from jax import lax
from jax.experimental import pallas as pl
from jax.experimental.pallas import tpu as pltpu
from jax.experimental.pallas import tpu_sc as plsc

N = 10000
E = 320000
D = 128

NC = 2
NS = 16
NW = NC * NS
EPW = E // NW
C = 40
CH = EPW // C
VPR = D // 16
RPT = 624
TAIL = N - NS * RPT

IA = 5
GA = 3
EA = 2
RI = 8
RG = 4
RE = 3
RM = 2


def _sc_body(x_hbm, src_hbm, dst_hbm, ea_hbm, out_hbm,
             sidx, didx, xr, ea, msg, acc, isem, gsem, esem, ssem):
    cid = lax.axis_index("c")
    sid = lax.axis_index("s")
    wid = cid * NS + sid
    base_e = wid * EPW

    base_r = sid * RPT

    def fire_idx(k):
        s = lax.rem(k, RI)
        pltpu.async_copy(src_hbm.at[pl.ds(base_e + k * C, C)], sidx.at[s],
                         isem.at[s])
        pltpu.async_copy(dst_hbm.at[pl.ds(base_e + k * C, C)], didx.at[s],
                         isem.at[s])

    def wait_idx(k):
        s = lax.rem(k, RI)
        pltpu.make_async_copy(src_hbm.at[pl.ds(base_e, C)], sidx.at[s],
                              isem.at[s]).wait()
        pltpu.make_async_copy(dst_hbm.at[pl.ds(base_e, C)], didx.at[s],
                              isem.at[s]).wait()

    def fire_gather(k):
        s = lax.rem(k, RG)
        pltpu.async_copy(x_hbm.at[sidx.at[lax.rem(k, RI)]], xr.at[s],
                         gsem.at[s])

    def wait_gather(k):
        s = lax.rem(k, RG)
        pltpu.make_async_copy(x_hbm.at[sidx.at[0]], xr.at[s],
                              gsem.at[s]).wait()

    def fire_ea(k):
        s = lax.rem(k, RE)
        pltpu.async_copy(ea_hbm.at[pl.ds(base_e + k * C, C)], ea.at[s],
                         esem.at[s])

    def wait_ea(k):
        s = lax.rem(k, RE)
        pltpu.make_async_copy(ea_hbm.at[pl.ds(base_e, C)], ea.at[s],
                              esem.at[s]).wait()

    def compute(k):
        sg = lax.rem(k, RG)
        se = lax.rem(k, RE)
        sm = lax.rem(k, RM)

        @plsc.parallel_loop(0, C, step=1, unroll=2)
        def _row(r):
            for j in range(VPR):
                sl = pl.ds(j * 16, 16)
                msg[sm, r, sl] = jnp.maximum(
                    xr[sg, r, sl] + ea[se, r, sl], 0.0)

    def fire_scat(k):
        sm = lax.rem(k, RM)
        pltpu.async_copy(msg.at[sm], acc.at[didx.at[lax.rem(k, RI)]],
                         ssem.at[sm], add=True)

    def wait_scat(k):
        sm = lax.rem(k, RM)
        pltpu.make_async_copy(msg.at[sm], acc.at[didx.at[0]],
                              ssem.at[sm]).wait()

    for c0 in range(IA):
        fire_idx(c0)
    for c0 in range(GA):
        wait_idx(c0)
        fire_gather(c0)
    for c0 in range(EA):
        fire_ea(c0)

    z0 = msg.at[0]

    def _zero_row(r, carry):
        for j in range(VPR):
            msg[0, r, pl.ds(j * 16, 16)] = jnp.zeros((16,), jnp.float32)
        return carry
    lax.fori_loop(0, C, _zero_row, 0)
    for t in range(RPT // C):
        pltpu.sync_copy(z0, acc.at[pl.ds(base_r + t * C, C)])
    rem = RPT % C
    if rem:
        pltpu.sync_copy(z0.at[pl.ds(0, rem)],
                        acc.at[pl.ds(base_r + (RPT // C) * C, rem)])

    @pl.when(sid == NS - 1)
    def _zero_tail():
        pltpu.sync_copy(z0.at[pl.ds(0, TAIL)],
                        acc.at[pl.ds(NS * RPT, TAIL)])

    plsc.subcore_barrier()

    def _step(c, carry):
        @pl.when(c + IA < CH)
        def _():
            fire_idx(c + IA)

        @pl.when(c + GA < CH)
        def _():
            wait_idx(c + GA)
            fire_gather(c + GA)

        @pl.when(c + EA < CH)
        def _():
            fire_ea(c + EA)
        wait_gather(c)
        wait_ea(c)

        @pl.when(c >= RM)
        def _():
            wait_scat(c - RM)
        compute(c)
        fire_scat(c)
        return carry

    lax.fori_loop(0, CH, _step, 0)
    wait_scat(CH - 2)
    wait_scat(CH - 1)

    plsc.subcore_barrier()
    pltpu.sync_copy(acc.at[pl.ds(base_r, RPT)],
                    out_hbm.at[cid, pl.ds(base_r, RPT)])

    @pl.when(sid == NS - 1)
    def _copy_tail():
        pltpu.sync_copy(acc.at[pl.ds(NS * RPT, TAIL)],
                        out_hbm.at[cid, pl.ds(NS * RPT, TAIL)])


_sc_aggr = functools.partial(
    pl.kernel,
    out_type=jax.ShapeDtypeStruct((NC, N, D), jnp.float32),
    mesh=plsc.VectorSubcoreMesh(core_axis_name="c", subcore_axis_name="s"),
    scratch_types=[
        pltpu.VMEM((RI, C), jnp.int32),
        pltpu.VMEM((RI, C), jnp.int32),
        pltpu.VMEM((RG, C, D), jnp.float32),
        pltpu.VMEM((RE, C, D), jnp.float32),
        pltpu.VMEM((RM, C, D), jnp.float32),
        pltpu.VMEM_SHARED((N, D), jnp.float32),
        pltpu.SemaphoreType.DMA((RI,)),
        pltpu.SemaphoreType.DMA((RG,)),
        pltpu.SemaphoreType.DMA((RE,)),
        pltpu.SemaphoreType.DMA((RM,)),
    ],
)(_sc_body)


def _tc_body(x_ref, a0_ref, a1_ref, w1_ref, b1_ref, w2_ref, b2_ref,
             g_ref, be_ref, o_ref):
    xb = x_ref[...]
    z = xb + a0_ref[...] + a1_ref[...]
    h1 = jnp.dot(z, w1_ref[...], preferred_element_type=jnp.float32)
    h1 = jnp.maximum(h1 + b1_ref[...], 0.0)
    h = jnp.dot(h1, w2_ref[...], preferred_element_type=jnp.float32)
    h = h + b2_ref[...]
    r = xb + jnp.maximum(h, 0.0)
    m = jnp.mean(r, axis=1, keepdims=True)
    cdev = r - m
    v = jnp.mean(cdev * cdev, axis=1, keepdims=True)
    o_ref[...] = cdev * lax.rsqrt(v + 1e-5) * g_ref[...] + be_ref[...]


BN = 1000


def _tc_mlp_ln(x, a0, a1, w1, b1, w2, b2, gamma, beta):
    row_spec = pl.BlockSpec((BN, D), lambda i: (i, 0))
    full_spec = pl.BlockSpec((D, D), lambda i: (0, 0))
    vec_spec = pl.BlockSpec((1, D), lambda i: (0, 0))
    return pl.pallas_call(
        _tc_body,
        grid=(N // BN,),
        in_specs=[row_spec, row_spec, row_spec, full_spec, vec_spec,
                  full_spec, vec_spec, vec_spec, vec_spec],
        out_specs=row_spec,
        out_shape=jax.ShapeDtypeStruct((N, D), jnp.float32),
    )(x, a0, a1, w1, b1.reshape(1, D), w2, b2.reshape(1, D),
      gamma.reshape(1, D), beta.reshape(1, D))


def kernel(x, edge_index, edge_attr, W1, b1, W2, b2, gamma, beta):
    src = edge_index[0].astype(jnp.int32)
    dst = edge_index[1].astype(jnp.int32)
    parts = _sc_aggr(x, src, dst, edge_attr)
    return _tc_mlp_ln(x, parts[0], parts[1], W1, b1, W2, b2, gamma, beta)
